# baseline (speedup 1.0000x reference)
import jax
import jax.numpy as jnp
from jax.experimental import pallas as pl
from jax.experimental.pallas import tpu as pltpu

TOP_K = 32
N_EXPERTS = 8
NEG = -1e30


def _lts_router_kernel(h_ref, lts_hbm, w_ref, b_ref, out_ref, ew_ref,
                       sel_vmem, sem):
    i = pl.program_id(0)

    @pl.when(i == 0)
    def _fetch_windows():
        for e in range(N_EXPERTS):
            pltpu.make_async_copy(
                lts_hbm.at[e, pl.ds(0, TOP_K)], sel_vmem.at[e], sem
            ).start()
        for e in range(N_EXPERTS):
            pltpu.make_async_copy(
                lts_hbm.at[e, pl.ds(0, TOP_K)], sel_vmem.at[e], sem
            ).wait()

    hb = h_ref[...]
    wt = w_ref[...]
    logits = jax.lax.dot_general(
        hb, wt, (((1,), (1,)), ((), ())),
        preferred_element_type=jnp.float32,
    ) + b_ref[...][None, :]

    m = jnp.max(logits, axis=-1, keepdims=True)
    el = jnp.exp(logits - m)
    ew_ref[...] = el / jnp.sum(el, axis=-1, keepdims=True)

    tb = logits.shape[0]
    iota_e = jax.lax.broadcasted_iota(jnp.int32, (tb, N_EXPERTS), 1)
    eidx = jnp.min(
        jnp.where(logits == m, iota_e, N_EXPERTS), axis=-1, keepdims=True
    )

    sel = sel_vmem[...].reshape(N_EXPERTS * TOP_K, hb.shape[1])
    sel16 = sel.astype(jnp.bfloat16)
    s = jax.lax.dot_general(
        hb.astype(jnp.bfloat16), sel16, (((1,), (1,)), ((), ())),
        preferred_element_type=jnp.float32,
    ) * (1.0 / jnp.sqrt(jnp.float32(hb.shape[1])))

    band = jax.lax.broadcasted_iota(jnp.int32, s.shape, 1) // TOP_K
    s = jnp.where(band == eidx, s, NEG)
    sm = jnp.max(s, axis=-1, keepdims=True)
    p = jnp.exp(s - sm)
    p = p / jnp.sum(p, axis=-1, keepdims=True)
    out_ref[...] = jax.lax.dot_general(
        p.astype(jnp.bfloat16), sel16, (((1,), (0,)), ((), ())),
        preferred_element_type=jnp.float32,
    )


def kernel(h, lts, W, b):
    bsz, t, d = h.shape
    n_lts = lts.shape[1]
    epe = n_lts // N_EXPERTS
    h2 = h.reshape(t, d)
    lts_r = lts.reshape(N_EXPERTS, epe, d)

    TB = 512
    grid = (t // TB,)

    result, ew = pl.pallas_call(
        _lts_router_kernel,
        grid=grid,
        in_specs=[
            pl.BlockSpec((TB, d), lambda i: (i, 0)),
            pl.BlockSpec(memory_space=pltpu.MemorySpace.HBM),
            pl.BlockSpec((N_EXPERTS, d), lambda i: (0, 0)),
            pl.BlockSpec((N_EXPERTS,), lambda i: (0,)),
        ],
        out_specs=[
            pl.BlockSpec((TB, d), lambda i: (i, 0)),
            pl.BlockSpec((TB, N_EXPERTS), lambda i: (i, 0)),
        ],
        out_shape=[
            jax.ShapeDtypeStruct((t, d), jnp.float32),
            jax.ShapeDtypeStruct((t, N_EXPERTS), jnp.float32),
        ],
        scratch_shapes=[
            pltpu.VMEM((N_EXPERTS, TOP_K, d), jnp.float32),
            pltpu.SemaphoreType.DMA,
        ],
    )(h2, lts_r, W, b)

    return (result.reshape(bsz, t, d), ew.reshape(bsz, t, N_EXPERTS))


# speedup vs baseline: 1.1752x; 1.1752x over previous
import jax
import jax.numpy as jnp
from jax.experimental import pallas as pl
from jax.experimental.pallas import tpu as pltpu

TOP_K = 32
N_EXPERTS = 8
NEG = -1e30


def _lts_router_kernel(h_ref, lts_ref, w_ref, b_ref, out_ref, ew_ref):
    hb = h_ref[...]
    wt = w_ref[...]
    logits = jax.lax.dot_general(
        hb, wt, (((1,), (1,)), ((), ())),
        preferred_element_type=jnp.float32,
    ) + b_ref[...][None, :]

    m = jnp.max(logits, axis=-1, keepdims=True)
    el = jnp.exp(logits - m)
    ew_ref[...] = el / jnp.sum(el, axis=-1, keepdims=True)

    tb = logits.shape[0]
    iota_e = jax.lax.broadcasted_iota(jnp.int32, (tb, N_EXPERTS), 1)
    eidx = jnp.min(
        jnp.where(logits == m, iota_e, N_EXPERTS), axis=-1, keepdims=True
    )

    sel = lts_ref[...].reshape(N_EXPERTS * TOP_K, hb.shape[1])
    sel16 = sel.astype(jnp.bfloat16)
    s = jax.lax.dot_general(
        hb.astype(jnp.bfloat16), sel16, (((1,), (1,)), ((), ())),
        preferred_element_type=jnp.float32,
    ) * (1.0 / jnp.sqrt(jnp.float32(hb.shape[1])))

    band = jax.lax.broadcasted_iota(jnp.int32, s.shape, 1) // TOP_K
    s = jnp.where(band == eidx, s, NEG)
    sm = jnp.max(s, axis=-1, keepdims=True)
    p = jnp.exp(s - sm)
    p = p / jnp.sum(p, axis=-1, keepdims=True)
    out_ref[...] = jax.lax.dot_general(
        p.astype(jnp.bfloat16), sel16, (((1,), (0,)), ((), ())),
        preferred_element_type=jnp.float32,
    )


def kernel(h, lts, W, b):
    bsz, t, d = h.shape
    n_lts = lts.shape[1]
    epe = n_lts // N_EXPERTS
    h2 = h.reshape(t, d)
    lts_r = lts.reshape(N_EXPERTS, epe, d)

    TB = 1024
    grid = (t // TB,)

    result, ew = pl.pallas_call(
        _lts_router_kernel,
        grid=grid,
        in_specs=[
            pl.BlockSpec((TB, d), lambda i: (i, 0)),
            pl.BlockSpec((N_EXPERTS, TOP_K, d), lambda i: (0, 0, 0)),
            pl.BlockSpec((N_EXPERTS, d), lambda i: (0, 0)),
            pl.BlockSpec((N_EXPERTS,), lambda i: (0,)),
        ],
        out_specs=[
            pl.BlockSpec((TB, d), lambda i: (i, 0)),
            pl.BlockSpec((TB, N_EXPERTS), lambda i: (i, 0)),
        ],
        out_shape=[
            jax.ShapeDtypeStruct((t, d), jnp.float32),
            jax.ShapeDtypeStruct((t, N_EXPERTS), jnp.float32),
        ],
        compiler_params=pltpu.CompilerParams(
            dimension_semantics=("parallel",),
        ),
    )(h2, lts_r, W, b)

    return (result.reshape(bsz, t, d), ew.reshape(bsz, t, N_EXPERTS))
